# baseline (speedup 1.0000x reference)
import functools

import jax
import jax.numpy as jnp
from jax import lax
from jax.experimental import pallas as pl
from jax.experimental.pallas import tpu as pltpu
from jax.experimental.pallas import tpu_sc as plsc

T = 8192
D = 4096
K = 8
E = 64
BT = 512
_NC = 2
_NS = 16
_NW = _NC * _NS
_TPW = T // _NW
_L = 16


def _eff_body(packed_hbm, scales_hbm, out_hbm,
              scales_v, idx_v, gate_v, prod_v, eff_v):
    wid = lax.axis_index("s") * _NC + lax.axis_index("c")
    base = wid * _TPW
    pltpu.sync_copy(scales_hbm, scales_v)
    pltpu.sync_copy(packed_hbm.at[pl.ds(base * K, _TPW * K)], idx_v)
    pltpu.sync_copy(packed_hbm.at[pl.ds(T * K + base * K, _TPW * K)], gate_v)

    def p1(j, _):
        i16 = plsc.bitcast(idx_v[pl.ds(j * _L, _L)], jnp.int32)
        g16 = gate_v[pl.ds(j * _L, _L)]
        s16 = plsc.load_gather(scales_v, [i16])
        prod_v[pl.ds(j * _L, _L)] = g16 * s16
        return _

    lax.fori_loop(0, _TPW * K // _L, p1, None, unroll=4)

    lane8 = jax.lax.iota(jnp.int32, _L) * K

    def p2(c, _):
        acc = jnp.zeros((_L,), jnp.float32)
        for k in range(K):
            acc = acc + plsc.load_gather(prod_v, [lane8 + (c * _L * K + k)])
        eff_v[pl.ds(c * _L, _L)] = acc
        return _

    lax.fori_loop(0, _TPW // _L, p2, None, unroll=1)
    pltpu.sync_copy(eff_v, out_hbm.at[pl.ds(base, _TPW)])


_eff_kernel = functools.partial(
    pl.kernel,
    mesh=plsc.VectorSubcoreMesh(core_axis_name="c", subcore_axis_name="s"),
    out_type=jax.ShapeDtypeStruct((T,), jnp.float32),
    scratch_types=[
        pltpu.VMEM((E,), jnp.float32),
        pltpu.VMEM((_TPW * K,), jnp.float32),
        pltpu.VMEM((_TPW * K,), jnp.float32),
        pltpu.VMEM((_TPW * K,), jnp.float32),
        pltpu.VMEM((_TPW,), jnp.float32),
    ],
    compiler_params=pltpu.CompilerParams(needs_layout_passes=False),
)(_eff_body)


def _scale_body(eff_ref, h_ref, out_ref):
    i = pl.program_id(0)
    e = eff_ref[pl.ds(i * (BT // 128), BT // 128), :]
    ii = lax.broadcasted_iota(jnp.int32, (128, 128), 0)
    jj = lax.broadcasted_iota(jnp.int32, (128, 128), 1)
    ident = (ii == jj).astype(jnp.float32)
    cols = lax.dot_general(ident, e, (((1,), (1,)), ((), ())),
                           precision=lax.Precision.HIGHEST,
                           preferred_element_type=jnp.float32)
    for a in range(BT // 128):
        out_ref[pl.ds(a * 128, 128), :] = (
            cols[:, a : a + 1] * h_ref[pl.ds(a * 128, 128), :]
        )


@jax.jit
def kernel(h, top_k_experts, expert_gate, scales):
    idx_bits = lax.bitcast_convert_type(
        top_k_experts.astype(jnp.int32), jnp.float32)
    packed = jnp.concatenate(
        [idx_bits.reshape(T * K), expert_gate.reshape(T * K)])
    eff = _eff_kernel(packed, scales)
    eff2 = eff.reshape(T // 128, 128)
    return pl.pallas_call(
        _scale_body,
        grid=(T // BT,),
        in_specs=[
            pl.BlockSpec((T // 128, 128), lambda i: (0, 0)),
            pl.BlockSpec((BT, D), lambda i: (i, 0)),
        ],
        out_specs=pl.BlockSpec((BT, D), lambda i: (i, 0)),
        out_shape=jax.ShapeDtypeStruct((T, D), jnp.float32),
    )(eff2, h)


# speedup vs baseline: 1.0822x; 1.0822x over previous
import functools

import jax
import jax.numpy as jnp
from jax import lax
from jax.experimental import pallas as pl
from jax.experimental.pallas import tpu as pltpu
from jax.experimental.pallas import tpu_sc as plsc

T = 8192
D = 4096
K = 8
E = 64
BT = 512
_NC = 2
_NS = 16
_NW = _NC * _NS
_TPW = T // _NW
_L = 16


def _eff_body(idx_hbm, gate_hbm, scales_hbm, out_hbm,
              scales_v, idx_v, gate_v, eff_v):
    wid = lax.axis_index("s") * _NC + lax.axis_index("c")
    base = wid * _TPW
    pltpu.sync_copy(scales_hbm, scales_v)
    pltpu.sync_copy(idx_hbm.at[:, pl.ds(base, _TPW)], idx_v)
    pltpu.sync_copy(gate_hbm.at[:, pl.ds(base, _TPW)], gate_v)

    def body(c, _):
        acc = jnp.zeros((_L,), jnp.float32)
        for k in range(K):
            i16 = idx_v[k, pl.ds(c * _L, _L)]
            g16 = gate_v[k, pl.ds(c * _L, _L)]
            acc = acc + g16 * plsc.load_gather(scales_v, [i16])
        eff_v[pl.ds(c * _L, _L)] = acc
        return _

    lax.fori_loop(0, _TPW // _L, body, None, unroll=2)
    pltpu.sync_copy(eff_v, out_hbm.at[pl.ds(base, _TPW)])


_eff_kernel = functools.partial(
    pl.kernel,
    mesh=plsc.VectorSubcoreMesh(core_axis_name="c", subcore_axis_name="s"),
    out_type=jax.ShapeDtypeStruct((T,), jnp.float32),
    scratch_types=[
        pltpu.VMEM((E,), jnp.float32),
        pltpu.VMEM((K, _TPW), jnp.int32),
        pltpu.VMEM((K, _TPW), jnp.float32),
        pltpu.VMEM((_TPW,), jnp.float32),
    ],
    compiler_params=pltpu.CompilerParams(needs_layout_passes=False),
)(_eff_body)


def _scale_body(eff_ref, h_ref, out_ref):
    i = pl.program_id(0)
    e = eff_ref[pl.ds(i * (BT // 128), BT // 128), :]
    ii = lax.broadcasted_iota(jnp.int32, (128, 128), 0)
    jj = lax.broadcasted_iota(jnp.int32, (128, 128), 1)
    ident = (ii == jj).astype(jnp.float32)
    cols = lax.dot_general(ident, e, (((1,), (1,)), ((), ())),
                           precision=lax.Precision.HIGHEST,
                           preferred_element_type=jnp.float32)
    for a in range(BT // 128):
        out_ref[pl.ds(a * 128, 128), :] = (
            cols[:, a : a + 1] * h_ref[pl.ds(a * 128, 128), :]
        )


@jax.jit
def kernel(h, top_k_experts, expert_gate, scales):
    idx_t = top_k_experts.astype(jnp.int32).T
    gate_t = expert_gate.T
    eff = _eff_kernel(idx_t, gate_t, scales)
    eff2 = eff.reshape(T // 128, 128)
    return pl.pallas_call(
        _scale_body,
        grid=(T // BT,),
        in_specs=[
            pl.BlockSpec((T // 128, 128), lambda i: (0, 0)),
            pl.BlockSpec((BT, D), lambda i: (i, 0)),
        ],
        out_specs=pl.BlockSpec((BT, D), lambda i: (i, 0)),
        out_shape=jax.ShapeDtypeStruct((T, D), jnp.float32),
    )(eff2, h)


# speedup vs baseline: 1.0919x; 1.0089x over previous
import functools

import jax
import jax.numpy as jnp
from jax import lax
from jax.experimental import pallas as pl
from jax.experimental.pallas import tpu as pltpu
from jax.experimental.pallas import tpu_sc as plsc

T = 8192
D = 4096
K = 8
E = 64
BT = 512
_NC = 2
_NS = 16
_NW = _NC * _NS
_TPW = T // _NW
_L = 16


def _eff_body(idx_hbm, gate_hbm, scales_hbm, out_hbm,
              scales_v, idx_v, gate_v, eff_v, sem_s, sem_i, sem_g):
    wid = lax.axis_index("s") * _NC + lax.axis_index("c")
    base = wid * _TPW
    cp_s = pltpu.async_copy(scales_hbm, scales_v, sem_s)
    cp_i = pltpu.async_copy(idx_hbm.at[:, pl.ds(base, _TPW)], idx_v, sem_i)
    cp_g = pltpu.async_copy(gate_hbm.at[:, pl.ds(base, _TPW)], gate_v, sem_g)
    cp_s.wait()
    cp_i.wait()
    cp_g.wait()

    def body(c, _):
        acc = jnp.zeros((_L,), jnp.float32)
        for k in range(K):
            i16 = idx_v[k, pl.ds(c * _L, _L)]
            g16 = gate_v[k, pl.ds(c * _L, _L)]
            acc = acc + g16 * plsc.load_gather(scales_v, [i16])
        eff_v[pl.ds(c * _L, _L)] = acc
        return _

    lax.fori_loop(0, _TPW // _L, body, None, unroll=4)
    pltpu.sync_copy(eff_v, out_hbm.at[pl.ds(base, _TPW)])


_eff_kernel = functools.partial(
    pl.kernel,
    mesh=plsc.VectorSubcoreMesh(core_axis_name="c", subcore_axis_name="s"),
    out_type=jax.ShapeDtypeStruct((T,), jnp.float32),
    scratch_types=[
        pltpu.VMEM((E,), jnp.float32),
        pltpu.VMEM((K, _TPW), jnp.int32),
        pltpu.VMEM((K, _TPW), jnp.float32),
        pltpu.VMEM((_TPW,), jnp.float32),
        pltpu.SemaphoreType.DMA,
        pltpu.SemaphoreType.DMA,
        pltpu.SemaphoreType.DMA,
    ],
    compiler_params=pltpu.CompilerParams(needs_layout_passes=False),
)(_eff_body)


def _scale_body(eff_ref, h_ref, out_ref):
    i = pl.program_id(0)
    e = eff_ref[pl.ds(i * (BT // 128), BT // 128), :]
    ii = lax.broadcasted_iota(jnp.int32, (128, 128), 0)
    jj = lax.broadcasted_iota(jnp.int32, (128, 128), 1)
    ident = (ii == jj).astype(jnp.float32)
    cols = lax.dot_general(ident, e, (((1,), (1,)), ((), ())),
                           precision=lax.Precision.HIGHEST,
                           preferred_element_type=jnp.float32)
    for a in range(BT // 128):
        out_ref[pl.ds(a * 128, 128), :] = (
            cols[:, a : a + 1] * h_ref[pl.ds(a * 128, 128), :]
        )


@jax.jit
def kernel(h, top_k_experts, expert_gate, scales):
    idx_t = top_k_experts.astype(jnp.int32).T
    gate_t = expert_gate.T
    eff = _eff_kernel(idx_t, gate_t, scales)
    eff2 = eff.reshape(T // 128, 128)
    return pl.pallas_call(
        _scale_body,
        grid=(T // BT,),
        in_specs=[
            pl.BlockSpec((T // 128, 128), lambda i: (0, 0)),
            pl.BlockSpec((BT, D), lambda i: (i, 0)),
        ],
        out_specs=pl.BlockSpec((BT, D), lambda i: (i, 0)),
        out_shape=jax.ShapeDtypeStruct((T, D), jnp.float32),
    )(eff2, h)


# speedup vs baseline: 1.1021x; 1.0093x over previous
import functools

import jax
import jax.numpy as jnp
from jax import lax
from jax.experimental import pallas as pl
from jax.experimental.pallas import tpu as pltpu
from jax.experimental.pallas import tpu_sc as plsc

T = 8192
D = 4096
K = 8
E = 64
BT = 512
_NC = 1
_NS = 16
_NW = _NC * _NS
_TPW = T // _NW
_L = 16


def _eff_body(idx_hbm, gate_hbm, scales_hbm, out_hbm,
              scales_v, idx_v, gate_v, eff_v, sem_s, sem_i, sem_g):
    wid = lax.axis_index("s") * _NC + lax.axis_index("c")
    base = wid * _TPW
    cp_s = pltpu.async_copy(scales_hbm, scales_v, sem_s)
    cp_i = pltpu.async_copy(idx_hbm.at[:, pl.ds(base, _TPW)], idx_v, sem_i)
    cp_g = pltpu.async_copy(gate_hbm.at[:, pl.ds(base, _TPW)], gate_v, sem_g)
    cp_s.wait()
    cp_i.wait()
    cp_g.wait()

    def body(c, _):
        acc = jnp.zeros((_L,), jnp.float32)
        for k in range(K):
            i16 = idx_v[k, pl.ds(c * _L, _L)]
            g16 = gate_v[k, pl.ds(c * _L, _L)]
            acc = acc + g16 * plsc.load_gather(scales_v, [i16])
        eff_v[pl.ds(c * _L, _L)] = acc
        return _

    lax.fori_loop(0, _TPW // _L, body, None, unroll=4)
    pltpu.sync_copy(eff_v, out_hbm.at[pl.ds(base, _TPW)])


_eff_kernel = functools.partial(
    pl.kernel,
    mesh=plsc.VectorSubcoreMesh(core_axis_name="c", subcore_axis_name="s", num_cores=1),
    out_type=jax.ShapeDtypeStruct((T,), jnp.float32),
    scratch_types=[
        pltpu.VMEM((E,), jnp.float32),
        pltpu.VMEM((K, _TPW), jnp.int32),
        pltpu.VMEM((K, _TPW), jnp.float32),
        pltpu.VMEM((_TPW,), jnp.float32),
        pltpu.SemaphoreType.DMA,
        pltpu.SemaphoreType.DMA,
        pltpu.SemaphoreType.DMA,
    ],
    compiler_params=pltpu.CompilerParams(needs_layout_passes=False),
)(_eff_body)


def _scale_body(eff_ref, h_ref, out_ref):
    i = pl.program_id(0)
    e = eff_ref[pl.ds(i * (BT // 128), BT // 128), :]
    ii = lax.broadcasted_iota(jnp.int32, (128, 128), 0)
    jj = lax.broadcasted_iota(jnp.int32, (128, 128), 1)
    ident = (ii == jj).astype(jnp.float32)
    cols = lax.dot_general(ident, e, (((1,), (1,)), ((), ())),
                           precision=lax.Precision.HIGHEST,
                           preferred_element_type=jnp.float32)
    for a in range(BT // 128):
        out_ref[pl.ds(a * 128, 128), :] = (
            cols[:, a : a + 1] * h_ref[pl.ds(a * 128, 128), :]
        )


@jax.jit
def kernel(h, top_k_experts, expert_gate, scales):
    idx_t = top_k_experts.astype(jnp.int32).T
    gate_t = expert_gate.T
    eff = _eff_kernel(idx_t, gate_t, scales)
    eff2 = eff.reshape(T // 128, 128)
    return pl.pallas_call(
        _scale_body,
        grid=(T // BT,),
        in_specs=[
            pl.BlockSpec((T // 128, 128), lambda i: (0, 0)),
            pl.BlockSpec((BT, D), lambda i: (i, 0)),
        ],
        out_specs=pl.BlockSpec((BT, D), lambda i: (i, 0)),
        out_shape=jax.ShapeDtypeStruct((T, D), jnp.float32),
    )(eff2, h)
